# baseline (speedup 1.0000x reference)
import functools

import jax
import jax.numpy as jnp
from jax import lax
from jax.experimental import pallas as pl
from jax.experimental.pallas import tpu as pltpu
from jax.experimental.pallas import tpu_sc as plsc

_VOCAB = 128
_HIDDEN = 128
_BATCH = 4
_SEQ = 8192
_B = _BATCH * _SEQ
_NC = 2
_NS = 16
_NW = _NC * _NS
_BPW = _B // _NW
_CH = 64
_NCHUNK = _BPW // _CH
_NBUF = 6


def _make_emb_kernel():
    mesh = plsc.VectorSubcoreMesh(core_axis_name="c", subcore_axis_name="s")
    out_s = jax.ShapeDtypeStruct((_B, _HIDDEN), jnp.float32)

    @functools.partial(
        pl.kernel,
        mesh=mesh,
        out_type=[out_s, out_s],
        scratch_types=[
            pltpu.VMEM((_BPW,), jnp.int32),
            pltpu.VMEM((_NBUF, _CH, _HIDDEN), jnp.float32),
            pltpu.VMEM_SHARED((_VOCAB, _HIDDEN), jnp.float32),
        ]
        + [pltpu.SemaphoreType.DMA] * (3 * _NBUF),
    )
    def emb(table_hbm, idx_hbm, out1_hbm, out2_hbm, idx_v, rows_v, table_sh,
            *sems):
        gsems = sems[:_NBUF]
        w1sems = sems[_NBUF:2 * _NBUF]
        w2sems = sems[2 * _NBUF:]
        sid = lax.axis_index("s")
        wid = sid * _NC + lax.axis_index("c")
        base = wid * _BPW

        @pl.when(sid == 0)
        def _():
            pltpu.sync_copy(table_hbm, table_sh)

        pltpu.sync_copy(
            idx_hbm.at[wid // 8, pl.ds((wid % 8) * _BPW, _BPW)], idx_v
        )
        plsc.subcore_barrier()

        def gstart(j):
            return pltpu.async_copy(
                table_sh.at[idx_v.at[pl.ds(j * _CH, _CH)]],
                rows_v.at[j % _NBUF],
                gsems[j % _NBUF],
            )

        def wstart(j):
            b = j % _NBUF
            dst = pl.ds(base + j * _CH, _CH)
            return (
                pltpu.async_copy(rows_v.at[b], out1_hbm.at[dst], w1sems[b]),
                pltpu.async_copy(rows_v.at[b], out2_hbm.at[dst], w2sems[b]),
            )

        gcp = {j: gstart(j) for j in range(_NBUF - 1)}
        wcp = {}
        for j in range(_NCHUNK):
            gcp[j].wait()
            wcp[j] = wstart(j)
            nj = j + _NBUF - 1
            if nj < _NCHUNK:
                if nj - _NBUF >= 0:
                    for c in wcp[nj - _NBUF]:
                        c.wait()
                gcp[nj] = gstart(nj)
        for j in range(_NCHUNK - _NBUF, _NCHUNK):
            if j >= 0:
                for c in wcp[j]:
                    c.wait()

    return emb


_emb = _make_emb_kernel()


def kernel(input_ids, embed_weight):
    ids = input_ids.astype(jnp.int32)
    out1, out2 = _emb(embed_weight, ids)
    return (
        out1.reshape(_BATCH, _SEQ, _HIDDEN),
        out2.reshape(_BATCH, _SEQ, _HIDDEN),
    )


# speedup vs baseline: 1.0093x; 1.0093x over previous
import functools

import jax
import jax.numpy as jnp
from jax import lax
from jax.experimental import pallas as pl
from jax.experimental.pallas import tpu as pltpu
from jax.experimental.pallas import tpu_sc as plsc

_VOCAB = 128
_HIDDEN = 128
_BATCH = 4
_SEQ = 8192
_B = _BATCH * _SEQ
_NC = 2
_NS = 16
_NW = _NC * _NS
_BPW = _B // _NW
_CH = 128
_NCHUNK = _BPW // _CH
_NBUF = 4


def _make_emb_kernel():
    mesh = plsc.VectorSubcoreMesh(core_axis_name="c", subcore_axis_name="s")
    out_s = jax.ShapeDtypeStruct((_B, _HIDDEN), jnp.float32)

    @functools.partial(
        pl.kernel,
        mesh=mesh,
        out_type=[out_s, out_s],
        scratch_types=[
            pltpu.VMEM((_BPW,), jnp.int32),
            pltpu.VMEM((_NBUF, _CH, _HIDDEN), jnp.float32),
            pltpu.VMEM_SHARED((_VOCAB, _HIDDEN), jnp.float32),
        ]
        + [pltpu.SemaphoreType.DMA] * (3 * _NBUF),
    )
    def emb(table_hbm, idx_hbm, out1_hbm, out2_hbm, idx_v, rows_v, table_sh,
            *sems):
        gsems = sems[:_NBUF]
        w1sems = sems[_NBUF:2 * _NBUF]
        w2sems = sems[2 * _NBUF:]
        sid = lax.axis_index("s")
        wid = sid * _NC + lax.axis_index("c")
        base = wid * _BPW

        @pl.when(sid == 0)
        def _():
            pltpu.sync_copy(table_hbm, table_sh)

        pltpu.sync_copy(
            idx_hbm.at[wid // 8, pl.ds((wid % 8) * _BPW, _BPW)], idx_v
        )
        plsc.subcore_barrier()

        def gstart(j):
            return pltpu.async_copy(
                table_sh.at[idx_v.at[pl.ds(j * _CH, _CH)]],
                rows_v.at[j % _NBUF],
                gsems[j % _NBUF],
            )

        def wstart(j):
            b = j % _NBUF
            dst = pl.ds(base + j * _CH, _CH)
            return (
                pltpu.async_copy(rows_v.at[b], out1_hbm.at[dst], w1sems[b]),
                pltpu.async_copy(rows_v.at[b], out2_hbm.at[dst], w2sems[b]),
            )

        gcp = {j: gstart(j) for j in range(_NBUF - 1)}
        wcp = {}
        for j in range(_NCHUNK):
            gcp[j].wait()
            wcp[j] = wstart(j)
            nj = j + _NBUF - 1
            if nj < _NCHUNK:
                if nj - _NBUF >= 0:
                    for c in wcp[nj - _NBUF]:
                        c.wait()
                gcp[nj] = gstart(nj)
        for j in range(_NCHUNK - _NBUF, _NCHUNK):
            if j >= 0:
                for c in wcp[j]:
                    c.wait()

    return emb


_emb = _make_emb_kernel()


def kernel(input_ids, embed_weight):
    ids = input_ids.astype(jnp.int32)
    out1, out2 = _emb(embed_weight, ids)
    return (
        out1.reshape(_BATCH, _SEQ, _HIDDEN),
        out2.reshape(_BATCH, _SEQ, _HIDDEN),
    )


# speedup vs baseline: 1.0197x; 1.0103x over previous
import functools

import jax
import jax.numpy as jnp
from jax import lax
from jax.experimental import pallas as pl
from jax.experimental.pallas import tpu as pltpu
from jax.experimental.pallas import tpu_sc as plsc

_VOCAB = 128
_HIDDEN = 128
_BATCH = 4
_SEQ = 8192
_B = _BATCH * _SEQ
_NC = 2
_NS = 16
_NW = _NC * _NS
_BPW = _B // _NW
_CH = 128
_NCHUNK = _BPW // _CH
_NBUF = 4


def _make_emb_kernel():
    mesh = plsc.VectorSubcoreMesh(core_axis_name="c", subcore_axis_name="s")
    out_s = jax.ShapeDtypeStruct((_B, _HIDDEN), jnp.float32)

    @functools.partial(
        pl.kernel,
        mesh=mesh,
        out_type=[out_s, out_s],
        scratch_types=[
            pltpu.VMEM((_BPW,), jnp.int32),
            pltpu.VMEM((_NBUF, _CH, _HIDDEN), jnp.float32),
            pltpu.VMEM_SHARED((_VOCAB, _HIDDEN), jnp.float32),
        ]
        + [pltpu.SemaphoreType.DMA] * (3 * _NBUF),
    )
    def emb(table_hbm, idx_hbm, out1_hbm, out2_hbm, idx_v, rows_v, table_sh,
            *sems):
        gsems = sems[:_NBUF]
        w1sems = sems[_NBUF:2 * _NBUF]
        w2sems = sems[2 * _NBUF:]
        sid = lax.axis_index("s")
        wid = sid * _NC + lax.axis_index("c")
        base = wid * _BPW

        rows_per_tile = _VOCAB // _NS
        tbl = pl.ds(sid * rows_per_tile, rows_per_tile)
        pltpu.sync_copy(table_hbm.at[tbl], table_sh.at[tbl])

        pltpu.sync_copy(
            idx_hbm.at[wid // 8, pl.ds((wid % 8) * _BPW, _BPW)], idx_v
        )
        plsc.subcore_barrier()

        def gstart(j):
            return pltpu.async_copy(
                table_sh.at[idx_v.at[pl.ds(j * _CH, _CH)]],
                rows_v.at[j % _NBUF],
                gsems[j % _NBUF],
            )

        def wstart(j):
            b = j % _NBUF
            dst = pl.ds(base + j * _CH, _CH)
            return (
                pltpu.async_copy(rows_v.at[b], out1_hbm.at[dst], w1sems[b]),
                pltpu.async_copy(rows_v.at[b], out2_hbm.at[dst], w2sems[b]),
            )

        gcp = {j: gstart(j) for j in range(_NBUF - 1)}
        wcp = {}
        for j in range(_NCHUNK):
            gcp[j].wait()
            wcp[j] = wstart(j)
            nj = j + _NBUF - 1
            if nj < _NCHUNK:
                if nj - _NBUF >= 0:
                    for c in wcp[nj - _NBUF]:
                        c.wait()
                gcp[nj] = gstart(nj)
        for j in range(_NCHUNK - _NBUF, _NCHUNK):
            if j >= 0:
                for c in wcp[j]:
                    c.wait()

    return emb


_emb = _make_emb_kernel()


def kernel(input_ids, embed_weight):
    ids = input_ids.astype(jnp.int32)
    out1, out2 = _emb(embed_weight, ids)
    return (
        out1.reshape(_BATCH, _SEQ, _HIDDEN),
        out2.reshape(_BATCH, _SEQ, _HIDDEN),
    )


# speedup vs baseline: 1.0254x; 1.0056x over previous
import functools

import jax
import jax.numpy as jnp
from jax import lax
from jax.experimental import pallas as pl
from jax.experimental.pallas import tpu as pltpu
from jax.experimental.pallas import tpu_sc as plsc

_VOCAB = 128
_HIDDEN = 128
_BATCH = 4
_SEQ = 8192
_B = _BATCH * _SEQ
_NC = 2
_NS = 16
_NW = _NC * _NS
_BPW = _B // _NW
_CH = 128
_NCHUNK = _BPW // _CH
_NBUF = 4


def _make_emb_kernel():
    mesh = plsc.VectorSubcoreMesh(core_axis_name="c", subcore_axis_name="s")
    out_s = jax.ShapeDtypeStruct((_B, _HIDDEN), jnp.float32)

    @functools.partial(
        pl.kernel,
        mesh=mesh,
        out_type=[out_s, out_s],
        scratch_types=[
            pltpu.VMEM((_BPW,), jnp.int32),
            pltpu.VMEM((_NBUF, _CH, _HIDDEN), jnp.float32),
            pltpu.VMEM_SHARED((_VOCAB, _HIDDEN), jnp.float32),
        ]
        + [pltpu.SemaphoreType.DMA] * (3 * _NBUF),
    )
    def emb(table_hbm, idx_hbm, out1_hbm, out2_hbm, idx_v, rows_v, table_sh,
            *sems):
        gsems = sems[:_NBUF]
        w1sems = sems[_NBUF:2 * _NBUF]
        w2sems = sems[2 * _NBUF:]
        sid = lax.axis_index("s")
        wid = sid * _NC + lax.axis_index("c")
        base = wid * _BPW

        rows_per_tile = _VOCAB // _NS
        tbl = pl.ds(sid * rows_per_tile, rows_per_tile)
        tcp = pltpu.async_copy(table_hbm.at[tbl], table_sh.at[tbl], gsems[0])
        icp = pltpu.async_copy(
            idx_hbm.at[wid // 8, pl.ds((wid % 8) * _BPW, _BPW)],
            idx_v,
            gsems[1],
        )
        tcp.wait()
        icp.wait()
        plsc.subcore_barrier()

        def gstart(j):
            return pltpu.async_copy(
                table_sh.at[idx_v.at[pl.ds(j * _CH, _CH)]],
                rows_v.at[j % _NBUF],
                gsems[j % _NBUF],
            )

        def wstart(j):
            b = j % _NBUF
            dst = pl.ds(base + j * _CH, _CH)
            return (
                pltpu.async_copy(rows_v.at[b], out1_hbm.at[dst], w1sems[b]),
                pltpu.async_copy(rows_v.at[b], out2_hbm.at[dst], w2sems[b]),
            )

        gcp = {j: gstart(j) for j in range(_NBUF - 1)}
        wcp = {}
        for j in range(_NCHUNK):
            gcp[j].wait()
            wcp[j] = wstart(j)
            nj = j + _NBUF - 1
            if nj < _NCHUNK:
                if nj - _NBUF >= 0:
                    for c in wcp[nj - _NBUF]:
                        c.wait()
                gcp[nj] = gstart(nj)
        for j in range(_NCHUNK - _NBUF, _NCHUNK):
            if j >= 0:
                for c in wcp[j]:
                    c.wait()

    return emb


_emb = _make_emb_kernel()


def kernel(input_ids, embed_weight):
    ids = input_ids.astype(jnp.int32)
    out1, out2 = _emb(embed_weight, ids)
    return (
        out1.reshape(_BATCH, _SEQ, _HIDDEN),
        out2.reshape(_BATCH, _SEQ, _HIDDEN),
    )


# speedup vs baseline: 1.0360x; 1.0104x over previous
import functools

import jax
import jax.numpy as jnp
from jax import lax
from jax.experimental import pallas as pl
from jax.experimental.pallas import tpu as pltpu
from jax.experimental.pallas import tpu_sc as plsc

_VOCAB = 128
_HIDDEN = 128
_BATCH = 4
_SEQ = 8192
_B = _BATCH * _SEQ
_NC = 2
_NS = 16
_NW = _NC * _NS
_BPW = _B // _NW
_CH = 128
_NCHUNK = _BPW // _CH
_NBUF = 4


def _make_emb_kernel():
    mesh = plsc.VectorSubcoreMesh(core_axis_name="c", subcore_axis_name="s")
    out_s = jax.ShapeDtypeStruct((_B, _HIDDEN), jnp.float32)

    @functools.partial(
        pl.kernel,
        mesh=mesh,
        out_type=[out_s, out_s],
        scratch_types=[
            pltpu.VMEM((_BPW,), jnp.int32),
            pltpu.VMEM((_NBUF, _CH, _HIDDEN), jnp.float32),
            pltpu.VMEM_SHARED((_VOCAB, _HIDDEN), jnp.float32),
        ]
        + [pltpu.SemaphoreType.DMA] * (3 * _NBUF),
    )
    def emb(table_hbm, idx_hbm, out1_hbm, out2_hbm, idx_v, rows_v, table_sh,
            *sems):
        gsems = sems[:_NBUF]
        w1sems = sems[_NBUF:2 * _NBUF]
        w2sems = sems[2 * _NBUF:]
        sid = lax.axis_index("s")
        wid = sid * _NC + lax.axis_index("c")
        base = wid * _BPW

        rows_per_tile = _VOCAB // _NS
        tbl = pl.ds(sid * rows_per_tile, rows_per_tile)
        tcp = pltpu.async_copy(table_hbm.at[tbl], table_sh.at[tbl], gsems[0])
        icp = pltpu.async_copy(
            idx_hbm.at[wid // 8, pl.ds((wid % 8) * _BPW, _BPW)],
            idx_v,
            gsems[1],
        )
        tcp.wait()
        icp.wait()
        plsc.subcore_barrier()

        sizes = [_CH // 2] + [_CH] * (_NCHUNK - 1) + [_CH // 2]
        offs = [0]
        for s in sizes[:-1]:
            offs.append(offs[-1] + s)
        nslice = len(sizes)

        def gstart(j):
            return pltpu.async_copy(
                table_sh.at[idx_v.at[pl.ds(offs[j], sizes[j])]],
                rows_v.at[j % _NBUF, pl.ds(0, sizes[j])],
                gsems[j % _NBUF],
            )

        def wstart(j):
            b = j % _NBUF
            src = rows_v.at[b, pl.ds(0, sizes[j])]
            dst = pl.ds(base + offs[j], sizes[j])
            return (
                pltpu.async_copy(src, out1_hbm.at[dst], w1sems[b]),
                pltpu.async_copy(src, out2_hbm.at[dst], w2sems[b]),
            )

        gcp = {j: gstart(j) for j in range(_NBUF - 1)}
        wcp = {}
        for j in range(nslice):
            gcp[j].wait()
            wcp[j] = wstart(j)
            nj = j + _NBUF - 1
            if nj < nslice:
                if nj - _NBUF >= 0:
                    for c in wcp[nj - _NBUF]:
                        c.wait()
                gcp[nj] = gstart(nj)
        for j in range(nslice - _NBUF, nslice):
            if j >= 0:
                for c in wcp[j]:
                    c.wait()

    return emb


_emb = _make_emb_kernel()


def kernel(input_ids, embed_weight):
    ids = input_ids.astype(jnp.int32)
    out1, out2 = _emb(embed_weight, ids)
    return (
        out1.reshape(_BATCH, _SEQ, _HIDDEN),
        out2.reshape(_BATCH, _SEQ, _HIDDEN),
    )


# speedup vs baseline: 1.0370x; 1.0009x over previous
import functools

import jax
import jax.numpy as jnp
from jax import lax
from jax.experimental import pallas as pl
from jax.experimental.pallas import tpu as pltpu
from jax.experimental.pallas import tpu_sc as plsc

_VOCAB = 128
_HIDDEN = 128
_BATCH = 4
_SEQ = 8192
_B = _BATCH * _SEQ
_NC = 2
_NS = 16
_NW = _NC * _NS
_BPW = _B // _NW
_CH = 128
_NCHUNK = _BPW // _CH
_NBUF = 4


def _make_emb_kernel():
    mesh = plsc.VectorSubcoreMesh(core_axis_name="c", subcore_axis_name="s")
    out_s = jax.ShapeDtypeStruct((_B, _HIDDEN), jnp.float32)

    @functools.partial(
        pl.kernel,
        mesh=mesh,
        out_type=[out_s, out_s],
        scratch_types=[
            pltpu.VMEM((_BPW,), jnp.int32),
            pltpu.VMEM((_NBUF, _CH, _HIDDEN), jnp.float32),
            pltpu.VMEM_SHARED((_VOCAB, _HIDDEN), jnp.float32),
        ]
        + [pltpu.SemaphoreType.DMA] * (3 * _NBUF),
    )
    def emb(table_hbm, idx_hbm, out1_hbm, out2_hbm, idx_v, rows_v, table_sh,
            *sems):
        gsems = sems[:_NBUF]
        w1sems = sems[_NBUF:2 * _NBUF]
        w2sems = sems[2 * _NBUF:]
        sid = lax.axis_index("s")
        wid = sid * _NC + lax.axis_index("c")
        base = wid * _BPW

        rows_per_tile = _VOCAB // _NS
        tbl = pl.ds(sid * rows_per_tile, rows_per_tile)
        tcp = pltpu.async_copy(table_hbm.at[tbl], table_sh.at[tbl], gsems[0])
        icp = pltpu.async_copy(
            idx_hbm.at[wid // 8, pl.ds((wid % 8) * _BPW, _BPW)],
            idx_v,
            gsems[1],
        )
        tcp.wait()
        icp.wait()
        plsc.subcore_barrier()

        sizes = [32, 96] + [_CH] * (_NCHUNK - 2) + [96, 32]
        offs = [0]
        for s in sizes[:-1]:
            offs.append(offs[-1] + s)
        nslice = len(sizes)

        def gstart(j):
            return pltpu.async_copy(
                table_sh.at[idx_v.at[pl.ds(offs[j], sizes[j])]],
                rows_v.at[j % _NBUF, pl.ds(0, sizes[j])],
                gsems[j % _NBUF],
            )

        def wstart(j):
            b = j % _NBUF
            src = rows_v.at[b, pl.ds(0, sizes[j])]
            dst = pl.ds(base + offs[j], sizes[j])
            return (
                pltpu.async_copy(src, out1_hbm.at[dst], w1sems[b]),
                pltpu.async_copy(src, out2_hbm.at[dst], w2sems[b]),
            )

        gcp = {j: gstart(j) for j in range(_NBUF - 1)}
        wcp = {}
        for j in range(nslice):
            gcp[j].wait()
            wcp[j] = wstart(j)
            nj = j + _NBUF - 1
            if nj < nslice:
                if nj - _NBUF >= 0:
                    for c in wcp[nj - _NBUF]:
                        c.wait()
                gcp[nj] = gstart(nj)
        for j in range(nslice - _NBUF, nslice):
            if j >= 0:
                for c in wcp[j]:
                    c.wait()

    return emb


_emb = _make_emb_kernel()


def kernel(input_ids, embed_weight):
    ids = input_ids.astype(jnp.int32)
    out1, out2 = _emb(embed_weight, ids)
    return (
        out1.reshape(_BATCH, _SEQ, _HIDDEN),
        out2.reshape(_BATCH, _SEQ, _HIDDEN),
    )
